# Initial kernel scaffold; baseline (speedup 1.0000x reference)
import jax
import jax.numpy as jnp
from jax.experimental import pallas as pl


def kernel(x, H, vgg, W, W_lin, b_hyper, W_se, b_se, W_pp, b_pp, W_m1, b_m1, W_m2, b_m2, W_at, b_at, prelu_a, W_tcn, b_tcn, W_res, b_res):
    raise NotImplementedError("write your pallas kernel here")


# speedup vs baseline: 7.0275x; 7.0275x over previous
import jax
import jax.numpy as jnp
from jax.experimental import pallas as pl

_L = 196
_DATT = 512
_DDOWN = 16
_BOTTLE = 4
_EMB = 10
_OUT_CH = 64
_T = 8
_N = 500
_NNZ = 4000
_NE = 500
_NPAD = 512
_PT = 32
_NP = _NPAD // _PT


def _ssa_kernel(coords_ref, vggT_ref, W_pp_ref, b_ppT_ref, W_m1f_ref,
                W_m1c_ref, W_se_ref, b_seT_ref, b_m1T_ref, W_m2_ref,
                b_m2T_ref, W_big_ref, b_at1_ref, out_ref):
    fpT = jnp.dot(W_pp_ref[...], vggT_ref[...]) + b_ppT_ref[...]
    AT = jnp.dot(W_m1f_ref[...], fpT)
    creT = jnp.dot(W_se_ref[...], coords_ref[...].T) + b_seT_ref[...]
    BpT = jnp.dot(W_m1c_ref[...], creT)
    pieces = []
    for p in range(_PT):
        pieces.append(AT + BpT[:, p:p + 1])
    hbig = jnp.concatenate(pieces, axis=1) + b_m1T_ref[...]
    hbig = jnp.maximum(hbig, 0.0)
    h2 = jnp.maximum(jnp.dot(W_m2_ref[...], hbig) + b_m2T_ref[...], 0.0)
    rows = []
    for p in range(_PT):
        h2p = h2[:, p * _L:(p + 1) * _L]
        rows.append(jnp.concatenate(
            [h2p[0:1, :], h2p[1:2, :], h2p[2:3, :], h2p[3:4, :]], axis=1))
    h2f = jnp.concatenate(rows, axis=0)
    logits = jnp.dot(h2f, W_big_ref[...]) + b_at1_ref[...]
    m = jnp.max(logits, axis=1, keepdims=True)
    w = jnp.exp(logits - m)
    w = w / jnp.sum(w, axis=1, keepdims=True)
    outs = []
    for c in range(_BOTTLE):
        hc = h2f[:, c * _L:(c + 1) * _L]
        outs.append(jnp.sum(hc * w, axis=1, keepdims=True))
    out_ref[...] = jnp.concatenate(outs, axis=1)


def _hconv_kernel(xT_ref, ssa_ref, H_ref, Ht_ref, Wx_ref, Ws_ref,
                  b_h1_ref, a_ref, out_ref):
    xt = xT_ref[0]
    xl = jnp.dot(xt, Wx_ref[...]) + jnp.dot(ssa_ref[...], Ws_ref[...])
    node_r = H_ref[0, 0, 0:1, :]
    edge_r = H_ref[0, 0, 1:2, :]
    node_c = Ht_ref[0, 0, :, 0:1]
    edge_c = Ht_ref[0, 0, :, 1:2]
    iota_s = jax.lax.broadcasted_iota(jnp.int32, (_NE, _NNZ), 0)
    iota_l = jax.lax.broadcasted_iota(jnp.int32, (_NNZ, _NE), 1)
    Gn = (iota_s == node_r).astype(jnp.float32)
    Ge = (iota_s == edge_r).astype(jnp.float32)
    GnT = (iota_l == node_c).astype(jnp.float32)
    GeT = (iota_l == edge_c).astype(jnp.float32)
    Ddeg = jnp.sum(Gn, axis=1, keepdims=True)
    Bdeg = jnp.sum(Ge, axis=1, keepdims=True)
    Dinv = jnp.where(Ddeg > 0, 1.0 / Ddeg, 0.0)
    Binv = jnp.where(Bdeg > 0, 1.0 / Bdeg, 0.0)
    gathered = jnp.dot(GnT, xl)
    e = Binv * jnp.dot(Ge, gathered)
    g2 = jnp.dot(GeT, e)
    nout = Dinv * jnp.dot(Gn, g2) + b_h1_ref[...]
    a = a_ref[...]
    out_ref[0] = jnp.where(nout >= 0, nout, a * nout)


def _tcn_kernel(hf_ref, x2_ref, Wt_ref, b_t1_ref, Wr_ref, b_r1_ref, out_ref):
    hf = hf_ref[...]
    z = jnp.zeros((_OUT_CH, _N), dtype=jnp.float32)
    hprev = jnp.concatenate([z, hf[:, :-_N]], axis=1)
    hnext = jnp.concatenate([hf[:, _N:], z], axis=1)
    conv = (jnp.dot(Wt_ref[0], hprev) + jnp.dot(Wt_ref[1], hf)
            + jnp.dot(Wt_ref[2], hnext) + b_t1_ref[...])
    res = jnp.dot(Wr_ref[...], x2_ref[...]) + b_r1_ref[...]
    out_ref[...] = conv + res


def kernel(x, H, vgg, W, W_lin, b_hyper, W_se, b_se, W_pp, b_pp, W_m1,
           b_m1, W_m2, b_m2, W_at, b_at, prelu_a, W_tcn, b_tcn, W_res,
           b_res):
    f32 = jnp.float32
    coords2 = jnp.pad(x[0, :, -1, :].T, ((0, _NPAD - _N), (0, 0)))
    vggT = vgg.reshape(_L, _DATT).T
    W_m1f = W_m1[:, :_DDOWN]
    W_m1c = W_m1[:, _DDOWN:]
    W_big = jnp.transpose(W_at.reshape(_L, _L, _BOTTLE), (2, 1, 0)
                          ).reshape(_L * _BOTTLE, _L)
    b_ppT = b_pp[:, None]
    b_seT = b_se[:, None]
    b_m1T = b_m1[:, None]
    b_m2T = b_m2[:, None]
    b_at1 = b_at[None, :]

    full = lambda arr: pl.BlockSpec(arr.shape, lambda i: (0,) * arr.ndim)
    ssa = pl.pallas_call(
        _ssa_kernel,
        grid=(_NP,),
        in_specs=[
            pl.BlockSpec((_PT, 2), lambda i: (i, 0)),
            full(vggT), full(W_pp), full(b_ppT), full(W_m1f),
            full(W_m1c), full(W_se), full(b_seT), full(b_m1T),
            full(W_m2), full(b_m2T), full(W_big), full(b_at1),
        ],
        out_specs=pl.BlockSpec((_PT, _BOTTLE), lambda i: (i, 0)),
        out_shape=jax.ShapeDtypeStruct((_NPAD, _BOTTLE), f32),
    )(coords2, vggT, W_pp, b_ppT, W_m1f, W_m1c, W_se, b_seT, b_m1T,
      W_m2, b_m2T, W_big, b_at1)
    ssa = ssa[:_N]

    xTt = jnp.transpose(x[0], (1, 2, 0))
    Ht = jnp.transpose(H, (0, 1, 3, 2))
    Wx = W_lin[:, :2].T
    Ws = W_lin[:, 2:].T
    b_h1 = b_hyper[None, :]
    a11 = prelu_a[:, None]

    fnf = pl.pallas_call(
        _hconv_kernel,
        grid=(_T,),
        in_specs=[
            pl.BlockSpec((1, _N, 2), lambda t: (t, 0, 0)),
            full(ssa),
            pl.BlockSpec((1, 1, 2, _NNZ), lambda t: (0, t, 0, 0)),
            pl.BlockSpec((1, 1, _NNZ, 2), lambda t: (0, t, 0, 0)),
            full(Wx), full(Ws), full(b_h1), full(a11),
        ],
        out_specs=pl.BlockSpec((1, _N, _OUT_CH), lambda t: (t, 0, 0)),
        out_shape=jax.ShapeDtypeStruct((_T, _N, _OUT_CH), f32),
    )(xTt, ssa, H, Ht, Wx, Ws, b_h1, a11)

    hf = fnf.reshape(_OUT_CH, _T * _N)
    x2 = x.reshape(2, _T * _N)
    Wt_s = jnp.transpose(W_tcn[:, :, :, 0], (2, 0, 1))
    Wr2 = W_res[:, :, 0, 0]

    out2 = pl.pallas_call(
        _tcn_kernel,
        out_shape=jax.ShapeDtypeStruct((_OUT_CH, _T * _N), f32),
    )(hf, x2, Wt_s, b_tcn[:, None], Wr2, b_res[:, None])

    return out2.reshape(1, _OUT_CH, _T, _N)


# speedup vs baseline: 10.9085x; 1.5523x over previous
import jax
import jax.numpy as jnp
from jax.experimental import pallas as pl

_L = 196
_DATT = 512
_DDOWN = 16
_BOTTLE = 4
_EMB = 10
_OUT_CH = 64
_T = 8
_N = 500
_NNZ = 4000
_NE = 500
_NPAD = 512
_PT = 32
_NP = _NPAD // _PT


def _ssa_kernel(coords_ref, vggT_ref, W_pp_ref, b_ppT_ref, W_m1b_ref,
                W_se_ref, b_seT_ref, E_ref, W_m2_ref,
                b_m2T_ref, W_big_ref, b_at1_ref, out_ref):
    fpT = jnp.dot(W_pp_ref[...], vggT_ref[...]) + b_ppT_ref[...]
    creT = jnp.dot(W_se_ref[...], coords_ref[...].T) + b_seT_ref[...]
    fp_tiled = jnp.concatenate([fpT] * _PT, axis=1)
    cre_exp = jnp.dot(creT, E_ref[...])
    ones_row = jnp.ones((1, _PT * _L), dtype=jnp.float32)
    X = jnp.concatenate([fp_tiled, cre_exp, ones_row], axis=0)
    hbig = jnp.maximum(jnp.dot(W_m1b_ref[...], X), 0.0)
    h2 = jnp.maximum(jnp.dot(W_m2_ref[...], hbig) + b_m2T_ref[...], 0.0)
    rows = []
    for p in range(_PT):
        h2p = h2[:, p * _L:(p + 1) * _L]
        rows.append(jnp.concatenate(
            [h2p[0:1, :], h2p[1:2, :], h2p[2:3, :], h2p[3:4, :]], axis=1))
    h2f = jnp.concatenate(rows, axis=0)
    logits = jnp.dot(h2f, W_big_ref[...]) + b_at1_ref[...]
    m = jnp.max(logits, axis=1, keepdims=True)
    w = jnp.exp(logits - m)
    w = w / jnp.sum(w, axis=1, keepdims=True)
    outs = []
    for c in range(_BOTTLE):
        hc = h2f[:, c * _L:(c + 1) * _L]
        outs.append(jnp.sum(hc * w, axis=1, keepdims=True))
    out_ref[...] = jnp.concatenate(outs, axis=1)


def _hconv_kernel(xT_ref, ssa_ref, H_ref, Ht_ref, Wx_ref, Ws_ref,
                  b_h1_ref, a_ref, out_ref):
    xt = xT_ref[0]
    xl = jnp.dot(xt, Wx_ref[...]) + jnp.dot(ssa_ref[...], Ws_ref[...])
    node_r = H_ref[0, 0, 0:1, :]
    edge_r = H_ref[0, 0, 1:2, :]
    node_c = Ht_ref[0, 0, :, 0:1]
    edge_c = Ht_ref[0, 0, :, 1:2]
    iota_s = jax.lax.broadcasted_iota(jnp.int32, (_NE, _NNZ), 0)
    iota_l = jax.lax.broadcasted_iota(jnp.int32, (_NNZ, _NE), 1)
    Gn = (iota_s == node_r).astype(jnp.float32)
    Ge = (iota_s == edge_r).astype(jnp.float32)
    GnT = (iota_l == node_c).astype(jnp.float32)
    GeT = (iota_l == edge_c).astype(jnp.float32)
    Ddeg = jnp.sum(Gn, axis=1, keepdims=True)
    Bdeg = jnp.sum(Ge, axis=1, keepdims=True)
    Dinv = jnp.where(Ddeg > 0, 1.0 / Ddeg, 0.0)
    Binv = jnp.where(Bdeg > 0, 1.0 / Bdeg, 0.0)
    gathered = jnp.dot(GnT, xl)
    e = Binv * jnp.dot(Ge, gathered)
    g2 = jnp.dot(GeT, e)
    nout = Dinv * jnp.dot(Gn, g2) + b_h1_ref[...]
    a = a_ref[...]
    out_ref[0] = jnp.where(nout >= 0, nout, a * nout)


def _tcn_kernel(hf_ref, x2_ref, Wt_ref, b_t1_ref, Wr_ref, b_r1_ref, out_ref):
    hf = hf_ref[...]
    z = jnp.zeros((_OUT_CH, _N), dtype=jnp.float32)
    hprev = jnp.concatenate([z, hf[:, :-_N]], axis=1)
    hnext = jnp.concatenate([hf[:, _N:], z], axis=1)
    conv = (jnp.dot(Wt_ref[0], hprev) + jnp.dot(Wt_ref[1], hf)
            + jnp.dot(Wt_ref[2], hnext) + b_t1_ref[...])
    res = jnp.dot(Wr_ref[...], x2_ref[...]) + b_r1_ref[...]
    out_ref[...] = conv + res


def kernel(x, H, vgg, W, W_lin, b_hyper, W_se, b_se, W_pp, b_pp, W_m1,
           b_m1, W_m2, b_m2, W_at, b_at, prelu_a, W_tcn, b_tcn, W_res,
           b_res):
    f32 = jnp.float32
    coords2 = jnp.pad(x[0, :, -1, :].T, ((0, _NPAD - _N), (0, 0)))
    vggT = vgg.reshape(_L, _DATT).T
    W_m1b = jnp.concatenate([W_m1, b_m1[:, None]], axis=1)
    E = jnp.kron(jnp.eye(_PT, dtype=f32), jnp.ones((1, _L), f32))
    W_big = jnp.transpose(W_at.reshape(_L, _L, _BOTTLE), (2, 1, 0)
                          ).reshape(_L * _BOTTLE, _L)
    b_ppT = b_pp[:, None]
    b_seT = b_se[:, None]
    b_m2T = b_m2[:, None]
    b_at1 = b_at[None, :]

    full = lambda arr: pl.BlockSpec(arr.shape, lambda i: (0,) * arr.ndim)
    ssa = pl.pallas_call(
        _ssa_kernel,
        grid=(_NP,),
        in_specs=[
            pl.BlockSpec((_PT, 2), lambda i: (i, 0)),
            full(vggT), full(W_pp), full(b_ppT), full(W_m1b),
            full(W_se), full(b_seT), full(E),
            full(W_m2), full(b_m2T), full(W_big), full(b_at1),
        ],
        out_specs=pl.BlockSpec((_PT, _BOTTLE), lambda i: (i, 0)),
        out_shape=jax.ShapeDtypeStruct((_NPAD, _BOTTLE), f32),
    )(coords2, vggT, W_pp, b_ppT, W_m1b, W_se, b_seT, E,
      W_m2, b_m2T, W_big, b_at1)
    ssa = ssa[:_N]

    xTt = jnp.transpose(x[0], (1, 2, 0))
    Ht = jnp.transpose(H, (0, 1, 3, 2))
    Wx = W_lin[:, :2].T
    Ws = W_lin[:, 2:].T
    b_h1 = b_hyper[None, :]
    a11 = prelu_a[:, None]

    fnf = pl.pallas_call(
        _hconv_kernel,
        grid=(_T,),
        in_specs=[
            pl.BlockSpec((1, _N, 2), lambda t: (t, 0, 0)),
            full(ssa),
            pl.BlockSpec((1, 1, 2, _NNZ), lambda t: (0, t, 0, 0)),
            pl.BlockSpec((1, 1, _NNZ, 2), lambda t: (0, t, 0, 0)),
            full(Wx), full(Ws), full(b_h1), full(a11),
        ],
        out_specs=pl.BlockSpec((1, _N, _OUT_CH), lambda t: (t, 0, 0)),
        out_shape=jax.ShapeDtypeStruct((_T, _N, _OUT_CH), f32),
    )(xTt, ssa, H, Ht, Wx, Ws, b_h1, a11)

    hf = fnf.reshape(_OUT_CH, _T * _N)
    x2 = x.reshape(2, _T * _N)
    Wt_s = jnp.transpose(W_tcn[:, :, :, 0], (2, 0, 1))
    Wr2 = W_res[:, :, 0, 0]

    out2 = pl.pallas_call(
        _tcn_kernel,
        out_shape=jax.ShapeDtypeStruct((_OUT_CH, _T * _N), f32),
    )(hf, x2, Wt_s, b_tcn[:, None], Wr2, b_res[:, None])

    return out2.reshape(1, _OUT_CH, _T, _N)


# speedup vs baseline: 10.9916x; 1.0076x over previous
import jax
import jax.numpy as jnp
from jax.experimental import pallas as pl

_L = 196
_DATT = 512
_DDOWN = 16
_BOTTLE = 4
_EMB = 10
_OUT_CH = 64
_T = 8
_N = 500
_NNZ = 4000
_NE = 500
_NPAD = 512
_PT = 32
_NP = _NPAD // _PT


def _ssa_kernel(coords_ref, vggT_ref, W_pp_ref, b_ppT_ref, W_m1b_ref,
                W_se_ref, b_seT_ref, E_ref, W_m2_ref,
                b_m2T_ref, W_big_ref, b_at1_ref, out_ref):
    fpT = jnp.dot(W_pp_ref[...], vggT_ref[...]) + b_ppT_ref[...]
    creT = jnp.dot(W_se_ref[...], coords_ref[...].T) + b_seT_ref[...]
    fp_tiled = jnp.concatenate([fpT] * _PT, axis=1)
    cre_exp = jnp.dot(creT, E_ref[...])
    ones_row = jnp.ones((1, _PT * _L), dtype=jnp.float32)
    X = jnp.concatenate([fp_tiled, cre_exp, ones_row], axis=0)
    hbig = jnp.maximum(
        jnp.dot(W_m1b_ref[...].astype(jnp.bfloat16), X.astype(jnp.bfloat16),
                preferred_element_type=jnp.float32), 0.0)
    h2 = jnp.maximum(
        jnp.dot(W_m2_ref[...].astype(jnp.bfloat16), hbig.astype(jnp.bfloat16),
                preferred_element_type=jnp.float32) + b_m2T_ref[...], 0.0)
    rows = []
    for p in range(_PT):
        h2p = h2[:, p * _L:(p + 1) * _L]
        rows.append(jnp.concatenate(
            [h2p[0:1, :], h2p[1:2, :], h2p[2:3, :], h2p[3:4, :]], axis=1))
    h2f = jnp.concatenate(rows, axis=0)
    logits = jnp.dot(h2f, W_big_ref[...]) + b_at1_ref[...]
    m = jnp.max(logits, axis=1, keepdims=True)
    w = jnp.exp(logits - m)
    w = w / jnp.sum(w, axis=1, keepdims=True)
    outs = []
    for c in range(_BOTTLE):
        hc = h2f[:, c * _L:(c + 1) * _L]
        outs.append(jnp.sum(hc * w, axis=1, keepdims=True))
    out_ref[...] = jnp.concatenate(outs, axis=1)


def _hconv_kernel(xT_ref, ssa_ref, H_ref, Ht_ref, Wx_ref, Ws_ref,
                  b_h1_ref, a_ref, out_ref):
    xt = xT_ref[0]
    xl = jnp.dot(xt, Wx_ref[...]) + jnp.dot(ssa_ref[...], Ws_ref[...])
    node_r = H_ref[0, 0, 0:1, :]
    edge_r = H_ref[0, 0, 1:2, :]
    node_c = Ht_ref[0, 0, :, 0:1]
    edge_c = Ht_ref[0, 0, :, 1:2]
    iota_s = jax.lax.broadcasted_iota(jnp.int32, (_NE, _NNZ), 0)
    iota_l = jax.lax.broadcasted_iota(jnp.int32, (_NNZ, _NE), 1)
    bf = jnp.bfloat16
    f32 = jnp.float32
    Gn = (iota_s == node_r).astype(bf)
    Ge = (iota_s == edge_r).astype(bf)
    GnT = (iota_l == node_c).astype(bf)
    GeT = (iota_l == edge_c).astype(bf)
    Ddeg = jnp.sum(Gn, axis=1, keepdims=True, dtype=f32)
    Bdeg = jnp.sum(Ge, axis=1, keepdims=True, dtype=f32)
    Dinv = jnp.where(Ddeg > 0, 1.0 / Ddeg, 0.0)
    Binv = jnp.where(Bdeg > 0, 1.0 / Bdeg, 0.0)
    gathered = jnp.dot(GnT, xl.astype(bf), preferred_element_type=f32)
    e = Binv * jnp.dot(Ge, gathered.astype(bf), preferred_element_type=f32)
    g2 = jnp.dot(GeT, e.astype(bf), preferred_element_type=f32)
    nout = (Dinv * jnp.dot(Gn, g2.astype(bf), preferred_element_type=f32)
            + b_h1_ref[...])
    a = a_ref[...]
    out_ref[0] = jnp.where(nout >= 0, nout, a * nout)


def _tcn_kernel(hf_ref, x2_ref, Wt_ref, b_t1_ref, Wr_ref, b_r1_ref, out_ref):
    hf = hf_ref[...]
    z = jnp.zeros((_OUT_CH, _N), dtype=jnp.float32)
    hprev = jnp.concatenate([z, hf[:, :-_N]], axis=1)
    hnext = jnp.concatenate([hf[:, _N:], z], axis=1)
    conv = (jnp.dot(Wt_ref[0], hprev) + jnp.dot(Wt_ref[1], hf)
            + jnp.dot(Wt_ref[2], hnext) + b_t1_ref[...])
    res = jnp.dot(Wr_ref[...], x2_ref[...]) + b_r1_ref[...]
    out_ref[...] = conv + res


def kernel(x, H, vgg, W, W_lin, b_hyper, W_se, b_se, W_pp, b_pp, W_m1,
           b_m1, W_m2, b_m2, W_at, b_at, prelu_a, W_tcn, b_tcn, W_res,
           b_res):
    f32 = jnp.float32
    coords2 = jnp.pad(x[0, :, -1, :].T, ((0, _NPAD - _N), (0, 0)))
    vggT = vgg.reshape(_L, _DATT).T
    W_m1b = jnp.concatenate([W_m1, b_m1[:, None]], axis=1)
    E = jnp.kron(jnp.eye(_PT, dtype=f32), jnp.ones((1, _L), f32))
    W_big = jnp.transpose(W_at.reshape(_L, _L, _BOTTLE), (2, 1, 0)
                          ).reshape(_L * _BOTTLE, _L)
    b_ppT = b_pp[:, None]
    b_seT = b_se[:, None]
    b_m2T = b_m2[:, None]
    b_at1 = b_at[None, :]

    full = lambda arr: pl.BlockSpec(arr.shape, lambda i: (0,) * arr.ndim)
    ssa = pl.pallas_call(
        _ssa_kernel,
        grid=(_NP,),
        in_specs=[
            pl.BlockSpec((_PT, 2), lambda i: (i, 0)),
            full(vggT), full(W_pp), full(b_ppT), full(W_m1b),
            full(W_se), full(b_seT), full(E),
            full(W_m2), full(b_m2T), full(W_big), full(b_at1),
        ],
        out_specs=pl.BlockSpec((_PT, _BOTTLE), lambda i: (i, 0)),
        out_shape=jax.ShapeDtypeStruct((_NPAD, _BOTTLE), f32),
    )(coords2, vggT, W_pp, b_ppT, W_m1b, W_se, b_seT, E,
      W_m2, b_m2T, W_big, b_at1)
    ssa = ssa[:_N]

    xTt = jnp.transpose(x[0], (1, 2, 0))
    Ht = jnp.transpose(H, (0, 1, 3, 2))
    Wx = W_lin[:, :2].T
    Ws = W_lin[:, 2:].T
    b_h1 = b_hyper[None, :]
    a11 = prelu_a[:, None]

    fnf = pl.pallas_call(
        _hconv_kernel,
        grid=(_T,),
        in_specs=[
            pl.BlockSpec((1, _N, 2), lambda t: (t, 0, 0)),
            full(ssa),
            pl.BlockSpec((1, 1, 2, _NNZ), lambda t: (0, t, 0, 0)),
            pl.BlockSpec((1, 1, _NNZ, 2), lambda t: (0, t, 0, 0)),
            full(Wx), full(Ws), full(b_h1), full(a11),
        ],
        out_specs=pl.BlockSpec((1, _N, _OUT_CH), lambda t: (t, 0, 0)),
        out_shape=jax.ShapeDtypeStruct((_T, _N, _OUT_CH), f32),
    )(xTt, ssa, H, Ht, Wx, Ws, b_h1, a11)

    hf = fnf.reshape(_OUT_CH, _T * _N)
    x2 = x.reshape(2, _T * _N)
    Wt_s = jnp.transpose(W_tcn[:, :, :, 0], (2, 0, 1))
    Wr2 = W_res[:, :, 0, 0]

    out2 = pl.pallas_call(
        _tcn_kernel,
        out_shape=jax.ShapeDtypeStruct((_OUT_CH, _T * _N), f32),
    )(hf, x2, Wt_s, b_tcn[:, None], Wr2, b_res[:, None])

    return out2.reshape(1, _OUT_CH, _T, _N)


# speedup vs baseline: 11.2779x; 1.0260x over previous
import jax
import jax.numpy as jnp
from jax.experimental import pallas as pl

_L = 196
_DATT = 512
_DDOWN = 16
_BOTTLE = 4
_EMB = 10
_OUT_CH = 64
_T = 8
_N = 500
_NNZ = 4000
_NE = 500
_NPAD = 512
_PT = 64
_NP = _NPAD // _PT


def _ssa_kernel(coords_ref, vggT_ref, W_pp_ref, b_ppT_ref, W_m1b_ref,
                W_se_ref, b_seT_ref, E_ref, W_m2_ref,
                b_m2T_ref, W_big_ref, b_at1_ref, out_ref):
    fpT = jnp.dot(W_pp_ref[...], vggT_ref[...]) + b_ppT_ref[...]
    creT = jnp.dot(W_se_ref[...], coords_ref[...].T) + b_seT_ref[...]
    fp_tiled = jnp.concatenate([fpT] * _PT, axis=1)
    cre_exp = jnp.dot(creT, E_ref[...])
    ones_row = jnp.ones((1, _PT * _L), dtype=jnp.float32)
    X = jnp.concatenate([fp_tiled, cre_exp, ones_row], axis=0)
    hbig = jnp.maximum(
        jnp.dot(W_m1b_ref[...].astype(jnp.bfloat16), X.astype(jnp.bfloat16),
                preferred_element_type=jnp.float32), 0.0)
    h2 = jnp.maximum(
        jnp.dot(W_m2_ref[...].astype(jnp.bfloat16), hbig.astype(jnp.bfloat16),
                preferred_element_type=jnp.float32) + b_m2T_ref[...], 0.0)
    rows = []
    for p in range(_PT):
        h2p = h2[:, p * _L:(p + 1) * _L]
        rows.append(jnp.concatenate(
            [h2p[0:1, :], h2p[1:2, :], h2p[2:3, :], h2p[3:4, :]], axis=1))
    h2f = jnp.concatenate(rows, axis=0)
    logits = jnp.dot(h2f, W_big_ref[...]) + b_at1_ref[...]
    m = jnp.max(logits, axis=1, keepdims=True)
    w = jnp.exp(logits - m)
    w = w / jnp.sum(w, axis=1, keepdims=True)
    outs = []
    for c in range(_BOTTLE):
        hc = h2f[:, c * _L:(c + 1) * _L]
        outs.append(jnp.sum(hc * w, axis=1, keepdims=True))
    out_ref[...] = jnp.concatenate(outs, axis=1)


def _hconv_kernel(xT_ref, ssa_ref, H_ref, Ht_ref, Wx_ref, Ws_ref,
                  b_h1_ref, a_ref, out_ref):
    xt = xT_ref[0]
    xl = jnp.dot(xt, Wx_ref[...]) + jnp.dot(ssa_ref[...], Ws_ref[...])
    node_r = H_ref[0, 0, 0:1, :]
    edge_r = H_ref[0, 0, 1:2, :]
    node_c = Ht_ref[0, 0, :, 0:1]
    edge_c = Ht_ref[0, 0, :, 1:2]
    iota_s = jax.lax.broadcasted_iota(jnp.int32, (_NE, _NNZ), 0)
    iota_l = jax.lax.broadcasted_iota(jnp.int32, (_NNZ, _NE), 1)
    bf = jnp.bfloat16
    f32 = jnp.float32
    Gn = (iota_s == node_r).astype(bf)
    Ge = (iota_s == edge_r).astype(bf)
    GnT = (iota_l == node_c).astype(bf)
    GeT = (iota_l == edge_c).astype(bf)
    Ddeg = jnp.sum(Gn, axis=1, keepdims=True, dtype=f32)
    Bdeg = jnp.sum(Ge, axis=1, keepdims=True, dtype=f32)
    Dinv = jnp.where(Ddeg > 0, 1.0 / Ddeg, 0.0)
    Binv = jnp.where(Bdeg > 0, 1.0 / Bdeg, 0.0)
    gathered = jnp.dot(GnT, xl.astype(bf), preferred_element_type=f32)
    e = Binv * jnp.dot(Ge, gathered.astype(bf), preferred_element_type=f32)
    g2 = jnp.dot(GeT, e.astype(bf), preferred_element_type=f32)
    nout = (Dinv * jnp.dot(Gn, g2.astype(bf), preferred_element_type=f32)
            + b_h1_ref[...])
    a = a_ref[...]
    out_ref[0] = jnp.where(nout >= 0, nout, a * nout)


def _tcn_kernel(hf_ref, x2_ref, Wt_ref, b_t1_ref, Wr_ref, b_r1_ref, out_ref):
    hf = hf_ref[...]
    z = jnp.zeros((_OUT_CH, _N), dtype=jnp.float32)
    hprev = jnp.concatenate([z, hf[:, :-_N]], axis=1)
    hnext = jnp.concatenate([hf[:, _N:], z], axis=1)
    conv = (jnp.dot(Wt_ref[0], hprev) + jnp.dot(Wt_ref[1], hf)
            + jnp.dot(Wt_ref[2], hnext) + b_t1_ref[...])
    res = jnp.dot(Wr_ref[...], x2_ref[...]) + b_r1_ref[...]
    out_ref[...] = conv + res


def kernel(x, H, vgg, W, W_lin, b_hyper, W_se, b_se, W_pp, b_pp, W_m1,
           b_m1, W_m2, b_m2, W_at, b_at, prelu_a, W_tcn, b_tcn, W_res,
           b_res):
    f32 = jnp.float32
    coords2 = jnp.pad(x[0, :, -1, :].T, ((0, _NPAD - _N), (0, 0)))
    vggT = vgg.reshape(_L, _DATT).T
    W_m1b = jnp.concatenate([W_m1, b_m1[:, None]], axis=1)
    E = jnp.kron(jnp.eye(_PT, dtype=f32), jnp.ones((1, _L), f32))
    W_big = jnp.transpose(W_at.reshape(_L, _L, _BOTTLE), (2, 1, 0)
                          ).reshape(_L * _BOTTLE, _L)
    b_ppT = b_pp[:, None]
    b_seT = b_se[:, None]
    b_m2T = b_m2[:, None]
    b_at1 = b_at[None, :]

    full = lambda arr: pl.BlockSpec(arr.shape, lambda i: (0,) * arr.ndim)
    ssa = pl.pallas_call(
        _ssa_kernel,
        grid=(_NP,),
        in_specs=[
            pl.BlockSpec((_PT, 2), lambda i: (i, 0)),
            full(vggT), full(W_pp), full(b_ppT), full(W_m1b),
            full(W_se), full(b_seT), full(E),
            full(W_m2), full(b_m2T), full(W_big), full(b_at1),
        ],
        out_specs=pl.BlockSpec((_PT, _BOTTLE), lambda i: (i, 0)),
        out_shape=jax.ShapeDtypeStruct((_NPAD, _BOTTLE), f32),
    )(coords2, vggT, W_pp, b_ppT, W_m1b, W_se, b_seT, E,
      W_m2, b_m2T, W_big, b_at1)
    ssa = ssa[:_N]

    xTt = jnp.transpose(x[0], (1, 2, 0))
    Ht = jnp.transpose(H, (0, 1, 3, 2))
    Wx = W_lin[:, :2].T
    Ws = W_lin[:, 2:].T
    b_h1 = b_hyper[None, :]
    a11 = prelu_a[:, None]

    fnf = pl.pallas_call(
        _hconv_kernel,
        grid=(_T,),
        in_specs=[
            pl.BlockSpec((1, _N, 2), lambda t: (t, 0, 0)),
            full(ssa),
            pl.BlockSpec((1, 1, 2, _NNZ), lambda t: (0, t, 0, 0)),
            pl.BlockSpec((1, 1, _NNZ, 2), lambda t: (0, t, 0, 0)),
            full(Wx), full(Ws), full(b_h1), full(a11),
        ],
        out_specs=pl.BlockSpec((1, _N, _OUT_CH), lambda t: (t, 0, 0)),
        out_shape=jax.ShapeDtypeStruct((_T, _N, _OUT_CH), f32),
    )(xTt, ssa, H, Ht, Wx, Ws, b_h1, a11)

    hf = fnf.reshape(_OUT_CH, _T * _N)
    x2 = x.reshape(2, _T * _N)
    Wt_s = jnp.transpose(W_tcn[:, :, :, 0], (2, 0, 1))
    Wr2 = W_res[:, :, 0, 0]

    out2 = pl.pallas_call(
        _tcn_kernel,
        out_shape=jax.ShapeDtypeStruct((_OUT_CH, _T * _N), f32),
    )(hf, x2, Wt_s, b_tcn[:, None], Wr2, b_res[:, None])

    return out2.reshape(1, _OUT_CH, _T, _N)
